# baseline (speedup 1.0000x reference)
import functools

import jax
import jax.numpy as jnp
from jax import lax
from jax.experimental import pallas as pl
from jax.experimental.pallas import tpu as pltpu
from jax.experimental.pallas import tpu_sc as plsc

N_NODES = 10000
F = 128
IN_F = 256
HID = 512
B = 4
S = 512
N_PAD = 10240
NC = 2
NS = 16
NW = NC * NS
CH_T = 80
EDGE_CAP = NW * CH_T * 128
ROWS_T = N_PAD // NS
NB = 2048
LBLK = 2048


def _leaky(v):
    return jnp.where(v >= 0, v, 0.01 * v)


GB = 8
CH0 = 96
CH1 = 64


def _sc_pass(rows_src, src2d, dst2d, ones_blk, zeros_blk, gather):
    mesh = plsc.VectorSubcoreMesh(core_axis_name="c", subcore_axis_name="s")

    @functools.partial(
        pl.kernel,
        mesh=mesh,
        out_type=jax.ShapeDtypeStruct((NC * N_PAD, F), jnp.float32),
        scratch_types=[
            pltpu.VMEM((2, GB, 128), jnp.int32),
            pltpu.VMEM((2, GB, 128), jnp.int32),
            pltpu.VMEM((CH_T, 128), jnp.int32),
            pltpu.VMEM((128, F), jnp.float32),
            pltpu.VMEM((128, F), jnp.float32),
            pltpu.VMEM_SHARED((N_PAD, F), jnp.float32),
            pltpu.SemaphoreType.DMA,
            pltpu.SemaphoreType.DMA,
            pltpu.SemaphoreType.DMA,
        ],
    )
    def k(rows_hbm, src_hbm, dst_hbm, ones_hbm, zero_hbm, out_hbm,
          srcb, dstb, dstv, buf0, buf1, accum, isem, gs0, gs1):
        bufs = (buf0, buf1)
        gsems = (gs0, gs1)
        c = lax.axis_index("c")
        s = lax.axis_index("s")
        pltpu.sync_copy(zero_hbm, accum.at[pl.ds(s * ROWS_T, ROWS_T)])
        if not gather:
            wid = s * NC + c
            base = wid * CH_T
            pltpu.sync_copy(dst_hbm.at[pl.ds(base, CH_T)], dstv)
            pltpu.sync_copy(ones_hbm, buf0)
            plsc.subcore_barrier()

            def body(j, carry):
                pltpu.sync_copy(buf0, accum.at[dstv.at[j]], add=True)
                return carry

            lax.fori_loop(0, CH_T, body, 0)
        else:
            ngb = jnp.where(c == 0, CH0 // GB, CH1 // GB)
            base = pl.multiple_of(
                jnp.where(c == 0, s * CH0, 16 * CH0 + s * CH1), GB)
            pltpu.sync_copy(src_hbm.at[pl.ds(base, GB)], srcb.at[0])
            pltpu.sync_copy(dst_hbm.at[pl.ds(base, GB)], dstb.at[0])
            pltpu.async_copy(src_hbm.at[pl.ds(base + GB, GB)], srcb.at[1],
                             isem)
            pltpu.async_copy(dst_hbm.at[pl.ds(base + GB, GB)], dstb.at[1],
                             isem)
            plsc.subcore_barrier()
            pltpu.async_copy(rows_hbm.at[srcb.at[0, 0]], buf0, gs0)
            pltpu.async_copy(rows_hbm.at[srcb.at[0, 1]], buf1, gs1)

            def chunk(sl, nsl, kk, prefetch):
                b = kk % 2
                pltpu.make_async_copy(
                    rows_hbm.at[srcb.at[0, 0]], bufs[b], gsems[b]).wait()
                pltpu.sync_copy(bufs[b], accum.at[dstb.at[sl, kk]],
                                add=True)
                if prefetch:
                    if kk < GB - 2:
                        sl2, r2 = sl, kk + 2
                    else:
                        if kk == GB - 2:
                            pltpu.make_async_copy(
                                src_hbm.at[pl.ds(base, GB)], srcb.at[0],
                                isem).wait()
                            pltpu.make_async_copy(
                                src_hbm.at[pl.ds(base, GB)], srcb.at[0],
                                isem).wait()
                        sl2, r2 = nsl, kk + 2 - GB
                    pltpu.async_copy(
                        rows_hbm.at[srcb.at[sl2, r2, pl.ds(0, 64)]],
                        bufs[b].at[pl.ds(0, 64)], gsems[b])
                    pltpu.async_copy(
                        rows_hbm.at[srcb.at[sl2, r2, pl.ds(64, 64)]],
                        bufs[b].at[pl.ds(64, 64)], gsems[b])

            def body(g, carry):
                sl = lax.rem(g, 2)
                nsl = 1 - sl
                pltpu.async_copy(
                    src_hbm.at[pl.ds(base + (g + 1) * GB, GB)],
                    srcb.at[nsl], isem)
                pltpu.async_copy(
                    dst_hbm.at[pl.ds(base + (g + 1) * GB, GB)],
                    dstb.at[nsl], isem)
                for kk in range(GB):
                    chunk(sl, nsl, kk, True)
                return carry

            for kk in range(GB):
                chunk(0, 1, kk, True)
            lax.fori_loop(1, ngb - 1, body, 0)
            g_last = ngb - 1
            sl_last = lax.rem(g_last, 2)
            for kk in range(GB):
                chunk(sl_last, 1 - sl_last, kk, kk < GB - 2)

        plsc.subcore_barrier()
        pltpu.sync_copy(
            accum.at[pl.ds(s * ROWS_T, ROWS_T)],
            out_hbm.at[pl.ds(c * N_PAD + s * ROWS_T, ROWS_T)])

    return k(rows_src, src2d, dst2d, ones_blk, zeros_blk)


def _deg_kernel(dega, degb, nodes, wg, dinv_out, y2_out):
    dinv = lax.rsqrt(dega[...] + degb[...] + 1.0)
    dinv_out[...] = dinv
    y2_out[...] = jnp.dot(nodes[...], wg[...],
                          preferred_element_type=jnp.float32) * dinv


def _layer_kernel(za, zb, y2p, dinv, bg, wg, y2n):
    h = _leaky(dinv[...] * (za[...] + zb[...] + y2p[...]) + bg[...])
    y2n[...] = jnp.dot(h, wg[...],
                       preferred_element_type=jnp.float32) * dinv[...]


def _h_kernel(x, w1, b1, h_out):
    h_out[...] = jnp.tanh(
        jnp.dot(x[0], w1[...], preferred_element_type=jnp.float32)
        + b1[...])[None]


def _att_kernel(h, x, nodes, w2, b2, out, k_scr):
    @pl.when(pl.program_id(1) == 0)
    def _():
        k_scr[...] = jnp.dot(nodes[...], w2[...],
                             preferred_element_type=jnp.float32) + b2[...]

    logits = lax.dot_general(h[0], k_scr[...], (((1,), (1,)), ((), ())),
                             preferred_element_type=jnp.float32)
    m = jnp.max(logits, axis=0, keepdims=True)
    p = jnp.exp(logits - m)
    att = p * (1.0 / jnp.sum(p, axis=0, keepdims=True))
    e = lax.dot_general(att, x[0], (((0,), (0,)), ((), ())),
                        preferred_element_type=jnp.float32)
    out[...] = jnp.maximum(e, 0.0)[None]


def _fin_kernel(e, nodes, za, zb, y2, dinv, bg2, wp, bp, out, proj_scr):
    @pl.when(pl.program_id(1) == 0)
    def _():
        g = _leaky(dinv[...] * (za[...] + zb[...] + y2[...]) + bg2[...])
        proj_scr[...] = (
            jnp.dot(nodes[...], wp[0:F, :], preferred_element_type=jnp.float32)
            + jnp.dot(g, wp[F:2 * F, :], preferred_element_type=jnp.float32)
            + bp[...])

    out[...] = e[...] * proj_scr[...][None]


def _nblk(i, j=0):
    return (i, j)


def kernel(x, nodes, adjacency, W1, b1, W2, b2,
           Wg0, bg0, Wg1, bg1, Wg2, bg2, Wp, bp):
    f32 = jnp.float32
    nodes_p = jnp.pad(nodes, ((0, N_PAD - N_NODES), (0, 0)))
    src = adjacency[0]
    dst = adjacency[1]
    pad_e = EDGE_CAP - src.shape[0]
    fill = jnp.full((pad_e,), N_PAD - 1, jnp.int32)
    src2d = jnp.concatenate([src, fill]).reshape(NW * CH_T, 128)
    dst2d = jnp.concatenate([dst, fill]).reshape(NW * CH_T, 128)
    zeros_blk = jnp.zeros((ROWS_T, F), f32)
    ones_blk = jnp.ones((128, F), f32)

    nb_grid = N_PAD // NB
    half = lambda i: (i + nb_grid, 0)
    row_spec = pl.BlockSpec((NB, F), lambda i: (i, 0))
    row_spec_hi = pl.BlockSpec((NB, F), half)
    mat_spec = lambda r, c: pl.BlockSpec((r, c), lambda i: (0, 0))

    degp = _sc_pass(zeros_blk, src2d, dst2d, ones_blk, zeros_blk, gather=False)
    dinv, y2 = pl.pallas_call(
        _deg_kernel,
        grid=(nb_grid,),
        in_specs=[row_spec, row_spec_hi, row_spec, mat_spec(F, F)],
        out_specs=[row_spec, row_spec],
        out_shape=[jax.ShapeDtypeStruct((N_PAD, F), f32),
                   jax.ShapeDtypeStruct((N_PAD, F), f32)],
    )(degp, degp, nodes_p, Wg0)

    h = pl.pallas_call(
        _h_kernel,
        grid=(B,),
        in_specs=[pl.BlockSpec((1, S, IN_F), lambda b: (b, 0, 0)),
                  mat_spec(IN_F, HID), mat_spec(1, HID)],
        out_specs=pl.BlockSpec((1, S, HID), lambda b: (b, 0, 0)),
        out_shape=jax.ShapeDtypeStruct((B, S, HID), f32),
    )(x, W1, b1.reshape(1, HID))
    nl_grid = N_PAD // LBLK
    lab_spec = pl.BlockSpec((LBLK, F), lambda l, b: (l, 0))
    lab_spec_hi = pl.BlockSpec((LBLK, F), lambda l, b: (l + nl_grid, 0))
    full2 = lambda r, c: pl.BlockSpec((r, c), lambda l, b: (0, 0))
    e_relu = pl.pallas_call(
        _att_kernel,
        grid=(nl_grid, B),
        in_specs=[pl.BlockSpec((1, S, HID), lambda l, b: (b, 0, 0)),
                  pl.BlockSpec((1, S, IN_F), lambda l, b: (b, 0, 0)),
                  lab_spec, full2(F, HID), full2(1, HID)],
        out_specs=pl.BlockSpec((1, LBLK, IN_F), lambda l, b: (b, l, 0)),
        out_shape=jax.ShapeDtypeStruct((B, N_PAD, IN_F), f32),
        scratch_shapes=[pltpu.VMEM((LBLK, HID), f32)],
    )(h, x, nodes_p, W2, b2.reshape(1, HID))

    for bg, wg in ((bg0, Wg1), (bg1, Wg2)):
        z = _sc_pass(y2, src2d, dst2d, ones_blk, zeros_blk, gather=True)
        y2 = pl.pallas_call(
            _layer_kernel,
            grid=(nb_grid,),
            in_specs=[row_spec, row_spec_hi, row_spec, row_spec,
                      mat_spec(1, F), mat_spec(F, F)],
            out_specs=row_spec,
            out_shape=jax.ShapeDtypeStruct((N_PAD, F), f32),
        )(z, z, y2, dinv, bg.reshape(1, F), wg)
    z2 = _sc_pass(y2, src2d, dst2d, ones_blk, zeros_blk, gather=True)

    out = pl.pallas_call(
        _fin_kernel,
        grid=(nl_grid, B),
        in_specs=[pl.BlockSpec((1, LBLK, IN_F), lambda l, b: (b, l, 0)),
                  lab_spec,
                  lab_spec, lab_spec_hi, lab_spec, lab_spec,
                  full2(1, F), full2(2 * F, IN_F), full2(1, IN_F)],
        out_specs=pl.BlockSpec((1, LBLK, IN_F), lambda l, b: (b, l, 0)),
        out_shape=jax.ShapeDtypeStruct((B, N_PAD, IN_F), f32),
        scratch_shapes=[pltpu.VMEM((LBLK, IN_F), f32)],
    )(e_relu, nodes_p, z2, z2, y2, dinv,
      bg2.reshape(1, F), Wp, bp.reshape(1, IN_F))
    return out[:, :N_NODES, :]


# speedup vs baseline: 1.0003x; 1.0003x over previous
import functools

import jax
import jax.numpy as jnp
from jax import lax
from jax.experimental import pallas as pl
from jax.experimental.pallas import tpu as pltpu
from jax.experimental.pallas import tpu_sc as plsc

N_NODES = 10000
F = 128
IN_F = 256
HID = 512
B = 4
S = 512
N_PAD = 10240
NC = 2
NS = 16
NW = NC * NS
CH_T = 80
EDGE_CAP = NW * CH_T * 128
ROWS_T = N_PAD // NS
NB = 2048
LBLK = 2048


def _leaky(v):
    return jnp.where(v >= 0, v, 0.01 * v)


GB = 8
CH0 = 96
CH1 = 64


def _sc_pass(rows_src, src2d, dst2d, ones_blk, zeros_blk, gather):
    mesh = plsc.VectorSubcoreMesh(core_axis_name="c", subcore_axis_name="s")

    @functools.partial(
        pl.kernel,
        mesh=mesh,
        out_type=jax.ShapeDtypeStruct((NC * N_PAD, F), jnp.float32),
        scratch_types=[
            pltpu.VMEM((2, GB, 128), jnp.int32),
            pltpu.VMEM((2, GB, 128), jnp.int32),
            pltpu.VMEM((CH_T, 128), jnp.int32),
            pltpu.VMEM((128, F), jnp.float32),
            pltpu.VMEM((128, F), jnp.float32),
            pltpu.VMEM_SHARED((N_PAD, F), jnp.float32),
            pltpu.SemaphoreType.DMA,
            pltpu.SemaphoreType.DMA,
            pltpu.SemaphoreType.DMA,
        ],
    )
    def k(rows_hbm, src_hbm, dst_hbm, ones_hbm, zero_hbm, out_hbm,
          srcb, dstb, dstv, buf0, buf1, accum, isem, gs0, gs1):
        bufs = (buf0, buf1)
        gsems = (gs0, gs1)
        c = lax.axis_index("c")
        s = lax.axis_index("s")
        pltpu.sync_copy(zero_hbm, accum.at[pl.ds(s * ROWS_T, ROWS_T)])
        if not gather:
            wid = s * NC + c
            base = wid * CH_T
            pltpu.sync_copy(dst_hbm.at[pl.ds(base, CH_T)], dstv)
            pltpu.sync_copy(ones_hbm, buf0)
            plsc.subcore_barrier()

            def body(j, carry):
                pltpu.sync_copy(buf0, accum.at[dstv.at[j]], add=True)
                return carry

            lax.fori_loop(0, CH_T, body, 0)
        else:
            ngb = jnp.where(c == 0, CH0 // GB, CH1 // GB)
            base = pl.multiple_of(
                jnp.where(c == 0, s * CH0, 16 * CH0 + s * CH1), GB)
            pltpu.sync_copy(src_hbm.at[pl.ds(base, GB)], srcb.at[0])
            pltpu.sync_copy(dst_hbm.at[pl.ds(base, GB)], dstb.at[0])
            pltpu.async_copy(src_hbm.at[pl.ds(base + GB, GB)], srcb.at[1],
                             isem)
            pltpu.async_copy(dst_hbm.at[pl.ds(base + GB, GB)], dstb.at[1],
                             isem)
            plsc.subcore_barrier()
            pltpu.async_copy(rows_hbm.at[srcb.at[0, 0]], buf0, gs0)
            pltpu.async_copy(rows_hbm.at[srcb.at[0, 1]], buf1, gs1)

            def chunk(sl, nsl, kk, prefetch):
                b = kk % 2
                pltpu.make_async_copy(
                    rows_hbm.at[srcb.at[0, 0]], bufs[b], gsems[b]).wait()
                pltpu.sync_copy(bufs[b], accum.at[dstb.at[sl, kk]],
                                add=True)
                if prefetch:
                    if kk < GB - 2:
                        sl2, r2 = sl, kk + 2
                    else:
                        if kk == GB - 2:
                            pltpu.make_async_copy(
                                src_hbm.at[pl.ds(base, GB)], srcb.at[0],
                                isem).wait()
                            pltpu.make_async_copy(
                                src_hbm.at[pl.ds(base, GB)], srcb.at[0],
                                isem).wait()
                        sl2, r2 = nsl, kk + 2 - GB
                    pltpu.async_copy(rows_hbm.at[srcb.at[sl2, r2]],
                                     bufs[b], gsems[b])

            def body(g, carry):
                sl = lax.rem(g, 2)
                nsl = 1 - sl
                pltpu.async_copy(
                    src_hbm.at[pl.ds(base + (g + 1) * GB, GB)],
                    srcb.at[nsl], isem)
                pltpu.async_copy(
                    dst_hbm.at[pl.ds(base + (g + 1) * GB, GB)],
                    dstb.at[nsl], isem)
                for kk in range(GB):
                    chunk(sl, nsl, kk, True)
                return carry

            for kk in range(GB):
                chunk(0, 1, kk, True)
            lax.fori_loop(1, ngb - 1, body, 0)
            g_last = ngb - 1
            sl_last = lax.rem(g_last, 2)
            for kk in range(GB):
                chunk(sl_last, 1 - sl_last, kk, kk < GB - 2)

        plsc.subcore_barrier()
        pltpu.sync_copy(
            accum.at[pl.ds(s * ROWS_T, ROWS_T)],
            out_hbm.at[pl.ds(c * N_PAD + s * ROWS_T, ROWS_T)])

    return k(rows_src, src2d, dst2d, ones_blk, zeros_blk)


def _deg_kernel(dega, degb, nodes, wg, dinv_out, y2_out):
    dinv = lax.rsqrt(dega[...] + degb[...] + 1.0)
    dinv_out[...] = dinv
    y2_out[...] = jnp.dot(nodes[...], wg[...],
                          preferred_element_type=jnp.float32) * dinv


def _layer_kernel(za, zb, y2p, dinv, bg, wg, y2n):
    h = _leaky(dinv[...] * (za[...] + zb[...] + y2p[...]) + bg[...])
    y2n[...] = jnp.dot(h, wg[...],
                       preferred_element_type=jnp.float32) * dinv[...]


def _h_kernel(x, w1, b1, h_out):
    h_out[...] = jnp.tanh(
        jnp.dot(x[0], w1[...], preferred_element_type=jnp.float32)
        + b1[...])[None]


def _att_kernel(h, x, nodes, w2, b2, out, k_scr):
    @pl.when(pl.program_id(1) == 0)
    def _():
        k_scr[...] = jnp.dot(nodes[...], w2[...],
                             preferred_element_type=jnp.float32) + b2[...]

    logits = lax.dot_general(h[0], k_scr[...], (((1,), (1,)), ((), ())),
                             preferred_element_type=jnp.float32)
    m = jnp.max(logits, axis=0, keepdims=True)
    p = jnp.exp(logits - m)
    att = p * (1.0 / jnp.sum(p, axis=0, keepdims=True))
    e = lax.dot_general(att, x[0], (((0,), (0,)), ((), ())),
                        preferred_element_type=jnp.float32)
    out[...] = jnp.maximum(e, 0.0)[None]


def _fin_kernel(e, nodes, za, zb, y2, dinv, bg2, wp, bp, out, proj_scr):
    @pl.when(pl.program_id(1) == 0)
    def _():
        g = _leaky(dinv[...] * (za[...] + zb[...] + y2[...]) + bg2[...])
        proj_scr[...] = (
            jnp.dot(nodes[...], wp[0:F, :], preferred_element_type=jnp.float32)
            + jnp.dot(g, wp[F:2 * F, :], preferred_element_type=jnp.float32)
            + bp[...])

    out[...] = e[...] * proj_scr[...][None]


def _nblk(i, j=0):
    return (i, j)


def kernel(x, nodes, adjacency, W1, b1, W2, b2,
           Wg0, bg0, Wg1, bg1, Wg2, bg2, Wp, bp):
    f32 = jnp.float32
    nodes_p = jnp.pad(nodes, ((0, N_PAD - N_NODES), (0, 0)))
    src = adjacency[0]
    dst = adjacency[1]
    pad_e = EDGE_CAP - src.shape[0]
    fill = jnp.full((pad_e,), N_PAD - 1, jnp.int32)
    src2d = jnp.concatenate([src, fill]).reshape(NW * CH_T, 128)
    dst2d = jnp.concatenate([dst, fill]).reshape(NW * CH_T, 128)
    zeros_blk = jnp.zeros((ROWS_T, F), f32)
    ones_blk = jnp.ones((128, F), f32)

    nb_grid = N_PAD // NB
    half = lambda i: (i + nb_grid, 0)
    row_spec = pl.BlockSpec((NB, F), lambda i: (i, 0))
    row_spec_hi = pl.BlockSpec((NB, F), half)
    mat_spec = lambda r, c: pl.BlockSpec((r, c), lambda i: (0, 0))

    degp = _sc_pass(zeros_blk, src2d, dst2d, ones_blk, zeros_blk, gather=False)
    dinv, y2 = pl.pallas_call(
        _deg_kernel,
        grid=(nb_grid,),
        in_specs=[row_spec, row_spec_hi, row_spec, mat_spec(F, F)],
        out_specs=[row_spec, row_spec],
        out_shape=[jax.ShapeDtypeStruct((N_PAD, F), f32),
                   jax.ShapeDtypeStruct((N_PAD, F), f32)],
    )(degp, degp, nodes_p, Wg0)

    h = pl.pallas_call(
        _h_kernel,
        grid=(B,),
        in_specs=[pl.BlockSpec((1, S, IN_F), lambda b: (b, 0, 0)),
                  mat_spec(IN_F, HID), mat_spec(1, HID)],
        out_specs=pl.BlockSpec((1, S, HID), lambda b: (b, 0, 0)),
        out_shape=jax.ShapeDtypeStruct((B, S, HID), f32),
    )(x, W1, b1.reshape(1, HID))
    nl_grid = N_PAD // LBLK
    lab_spec = pl.BlockSpec((LBLK, F), lambda l, b: (l, 0))
    lab_spec_hi = pl.BlockSpec((LBLK, F), lambda l, b: (l + nl_grid, 0))
    full2 = lambda r, c: pl.BlockSpec((r, c), lambda l, b: (0, 0))
    e_relu = pl.pallas_call(
        _att_kernel,
        grid=(nl_grid, B),
        in_specs=[pl.BlockSpec((1, S, HID), lambda l, b: (b, 0, 0)),
                  pl.BlockSpec((1, S, IN_F), lambda l, b: (b, 0, 0)),
                  lab_spec, full2(F, HID), full2(1, HID)],
        out_specs=pl.BlockSpec((1, LBLK, IN_F), lambda l, b: (b, l, 0)),
        out_shape=jax.ShapeDtypeStruct((B, N_PAD, IN_F), f32),
        scratch_shapes=[pltpu.VMEM((LBLK, HID), f32)],
    )(h, x, nodes_p, W2, b2.reshape(1, HID))

    for bg, wg in ((bg0, Wg1), (bg1, Wg2)):
        z = _sc_pass(y2, src2d, dst2d, ones_blk, zeros_blk, gather=True)
        y2 = pl.pallas_call(
            _layer_kernel,
            grid=(nb_grid,),
            in_specs=[row_spec, row_spec_hi, row_spec, row_spec,
                      mat_spec(1, F), mat_spec(F, F)],
            out_specs=row_spec,
            out_shape=jax.ShapeDtypeStruct((N_PAD, F), f32),
        )(z, z, y2, dinv, bg.reshape(1, F), wg)
    z2 = _sc_pass(y2, src2d, dst2d, ones_blk, zeros_blk, gather=True)

    out = pl.pallas_call(
        _fin_kernel,
        grid=(nl_grid, B),
        in_specs=[pl.BlockSpec((1, LBLK, IN_F), lambda l, b: (b, l, 0)),
                  lab_spec,
                  lab_spec, lab_spec_hi, lab_spec, lab_spec,
                  full2(1, F), full2(2 * F, IN_F), full2(1, IN_F)],
        out_specs=pl.BlockSpec((1, LBLK, IN_F), lambda l, b: (b, l, 0)),
        out_shape=jax.ShapeDtypeStruct((B, N_PAD, IN_F), f32),
        scratch_shapes=[pltpu.VMEM((LBLK, IN_F), f32)],
    )(e_relu, nodes_p, z2, z2, y2, dinv,
      bg2.reshape(1, F), Wp, bp.reshape(1, IN_F))
    return out[:, :N_NODES, :]


# speedup vs baseline: 1.0015x; 1.0013x over previous
import functools

import jax
import jax.numpy as jnp
from jax import lax
from jax.experimental import pallas as pl
from jax.experimental.pallas import tpu as pltpu
from jax.experimental.pallas import tpu_sc as plsc

N_NODES = 10000
F = 128
IN_F = 256
HID = 512
B = 4
S = 512
N_PAD = 10240
NC = 2
NS = 16
NW = NC * NS
CH_T = 80
EDGE_CAP = NW * CH_T * 128
ROWS_T = N_PAD // NS
NB = 2048
LBLK = 2048


def _leaky(v):
    return jnp.where(v >= 0, v, 0.01 * v)


GB = 8
CH0 = 96
CH1 = 64


def _sc_pass(rows_src, src2d, dst2d, ones_blk, zeros_blk, gather):
    mesh = plsc.VectorSubcoreMesh(core_axis_name="c", subcore_axis_name="s")

    @functools.partial(
        pl.kernel,
        mesh=mesh,
        out_type=jax.ShapeDtypeStruct((NC * N_PAD, F), jnp.float32),
        scratch_types=[
            pltpu.VMEM((2, GB, 128), jnp.int32),
            pltpu.VMEM((2, GB, 128), jnp.int32),
            pltpu.VMEM((CH_T, 128), jnp.int32),
            pltpu.VMEM((128, F), jnp.float32),
            pltpu.VMEM((128, F), jnp.float32),
            pltpu.VMEM_SHARED((N_PAD, F), jnp.float32),
            pltpu.SemaphoreType.DMA,
            pltpu.SemaphoreType.DMA,
            pltpu.SemaphoreType.DMA,
        ],
    )
    def k(rows_hbm, src_hbm, dst_hbm, ones_hbm, zero_hbm, out_hbm,
          srcb, dstb, dstv, buf0, buf1, accum, isem, gs0, gs1):
        bufs = (buf0, buf1)
        gsems = (gs0, gs1)
        c = lax.axis_index("c")
        s = lax.axis_index("s")
        pltpu.sync_copy(zero_hbm, accum.at[pl.ds(s * ROWS_T, ROWS_T)])
        if not gather:
            wid = s * NC + c
            base = wid * CH_T
            pltpu.sync_copy(dst_hbm.at[pl.ds(base, CH_T)], dstv)
            pltpu.sync_copy(ones_hbm, buf0)
            plsc.subcore_barrier()

            def body(j, carry):
                pltpu.async_copy(buf0, accum.at[dstv.at[j]], isem, add=True)
                return carry

            lax.fori_loop(0, CH_T, body, 0)

            def drain(j, carry):
                pltpu.make_async_copy(buf0, accum.at[dstv.at[0]],
                                      isem).wait()
                return carry

            lax.fori_loop(0, CH_T, drain, 0)
        else:
            ngb = jnp.where(c == 0, CH0 // GB, CH1 // GB)
            base = pl.multiple_of(
                jnp.where(c == 0, s * CH0, 16 * CH0 + s * CH1), GB)
            pltpu.sync_copy(src_hbm.at[pl.ds(base, GB)], srcb.at[0])
            pltpu.sync_copy(dst_hbm.at[pl.ds(base, GB)], dstb.at[0])
            pltpu.async_copy(src_hbm.at[pl.ds(base + GB, GB)], srcb.at[1],
                             isem)
            pltpu.async_copy(dst_hbm.at[pl.ds(base + GB, GB)], dstb.at[1],
                             isem)
            plsc.subcore_barrier()
            pltpu.async_copy(rows_hbm.at[srcb.at[0, 0]], buf0, gs0)
            pltpu.async_copy(rows_hbm.at[srcb.at[0, 1]], buf1, gs1)

            def chunk(sl, nsl, kk, prefetch):
                b = kk % 2
                pltpu.make_async_copy(
                    rows_hbm.at[srcb.at[0, 0]], bufs[b], gsems[b]).wait()
                pltpu.sync_copy(bufs[b], accum.at[dstb.at[sl, kk]],
                                add=True)
                if prefetch:
                    if kk < GB - 2:
                        sl2, r2 = sl, kk + 2
                    else:
                        if kk == GB - 2:
                            pltpu.make_async_copy(
                                src_hbm.at[pl.ds(base, GB)], srcb.at[0],
                                isem).wait()
                            pltpu.make_async_copy(
                                src_hbm.at[pl.ds(base, GB)], srcb.at[0],
                                isem).wait()
                        sl2, r2 = nsl, kk + 2 - GB
                    pltpu.async_copy(rows_hbm.at[srcb.at[sl2, r2]],
                                     bufs[b], gsems[b])

            def body(g, carry):
                sl = lax.rem(g, 2)
                nsl = 1 - sl
                pltpu.async_copy(
                    src_hbm.at[pl.ds(base + (g + 1) * GB, GB)],
                    srcb.at[nsl], isem)
                pltpu.async_copy(
                    dst_hbm.at[pl.ds(base + (g + 1) * GB, GB)],
                    dstb.at[nsl], isem)
                for kk in range(GB):
                    chunk(sl, nsl, kk, True)
                return carry

            for kk in range(GB):
                chunk(0, 1, kk, True)
            lax.fori_loop(1, ngb - 1, body, 0)
            g_last = ngb - 1
            sl_last = lax.rem(g_last, 2)
            for kk in range(GB):
                chunk(sl_last, 1 - sl_last, kk, kk < GB - 2)

        plsc.subcore_barrier()
        pltpu.sync_copy(
            accum.at[pl.ds(s * ROWS_T, ROWS_T)],
            out_hbm.at[pl.ds(c * N_PAD + s * ROWS_T, ROWS_T)])

    return k(rows_src, src2d, dst2d, ones_blk, zeros_blk)


def _deg_kernel(dega, degb, nodes, wg, dinv_out, y2_out):
    dinv = lax.rsqrt(dega[...] + degb[...] + 1.0)
    dinv_out[...] = dinv
    y2_out[...] = jnp.dot(nodes[...], wg[...],
                          preferred_element_type=jnp.float32) * dinv


def _layer_kernel(za, zb, y2p, dinv, bg, wg, y2n):
    h = _leaky(dinv[...] * (za[...] + zb[...] + y2p[...]) + bg[...])
    y2n[...] = jnp.dot(h, wg[...],
                       preferred_element_type=jnp.float32) * dinv[...]


def _h_kernel(x, w1, b1, h_out):
    h_out[...] = jnp.tanh(
        jnp.dot(x[0], w1[...], preferred_element_type=jnp.float32)
        + b1[...])[None]


def _att_kernel(h, x, nodes, w2, b2, out, k_scr):
    @pl.when(pl.program_id(1) == 0)
    def _():
        k_scr[...] = jnp.dot(nodes[...], w2[...],
                             preferred_element_type=jnp.float32) + b2[...]

    logits = lax.dot_general(h[0], k_scr[...], (((1,), (1,)), ((), ())),
                             preferred_element_type=jnp.float32)
    m = jnp.max(logits, axis=0, keepdims=True)
    p = jnp.exp(logits - m)
    att = p * (1.0 / jnp.sum(p, axis=0, keepdims=True))
    e = lax.dot_general(att, x[0], (((0,), (0,)), ((), ())),
                        preferred_element_type=jnp.float32)
    out[...] = jnp.maximum(e, 0.0)[None]


def _fin_kernel(e, nodes, za, zb, y2, dinv, bg2, wp, bp, out, proj_scr):
    @pl.when(pl.program_id(1) == 0)
    def _():
        g = _leaky(dinv[...] * (za[...] + zb[...] + y2[...]) + bg2[...])
        proj_scr[...] = (
            jnp.dot(nodes[...], wp[0:F, :], preferred_element_type=jnp.float32)
            + jnp.dot(g, wp[F:2 * F, :], preferred_element_type=jnp.float32)
            + bp[...])

    out[...] = e[...] * proj_scr[...][None]


def _nblk(i, j=0):
    return (i, j)


def kernel(x, nodes, adjacency, W1, b1, W2, b2,
           Wg0, bg0, Wg1, bg1, Wg2, bg2, Wp, bp):
    f32 = jnp.float32
    nodes_p = jnp.pad(nodes, ((0, N_PAD - N_NODES), (0, 0)))
    src = adjacency[0]
    dst = adjacency[1]
    pad_e = EDGE_CAP - src.shape[0]
    fill = jnp.full((pad_e,), N_PAD - 1, jnp.int32)
    src2d = jnp.concatenate([src, fill]).reshape(NW * CH_T, 128)
    dst2d = jnp.concatenate([dst, fill]).reshape(NW * CH_T, 128)
    zeros_blk = jnp.zeros((ROWS_T, F), f32)
    ones_blk = jnp.ones((128, F), f32)

    nb_grid = N_PAD // NB
    half = lambda i: (i + nb_grid, 0)
    row_spec = pl.BlockSpec((NB, F), lambda i: (i, 0))
    row_spec_hi = pl.BlockSpec((NB, F), half)
    mat_spec = lambda r, c: pl.BlockSpec((r, c), lambda i: (0, 0))

    degp = _sc_pass(zeros_blk, src2d, dst2d, ones_blk, zeros_blk, gather=False)
    dinv, y2 = pl.pallas_call(
        _deg_kernel,
        grid=(nb_grid,),
        in_specs=[row_spec, row_spec_hi, row_spec, mat_spec(F, F)],
        out_specs=[row_spec, row_spec],
        out_shape=[jax.ShapeDtypeStruct((N_PAD, F), f32),
                   jax.ShapeDtypeStruct((N_PAD, F), f32)],
    )(degp, degp, nodes_p, Wg0)

    h = pl.pallas_call(
        _h_kernel,
        grid=(B,),
        in_specs=[pl.BlockSpec((1, S, IN_F), lambda b: (b, 0, 0)),
                  mat_spec(IN_F, HID), mat_spec(1, HID)],
        out_specs=pl.BlockSpec((1, S, HID), lambda b: (b, 0, 0)),
        out_shape=jax.ShapeDtypeStruct((B, S, HID), f32),
    )(x, W1, b1.reshape(1, HID))
    nl_grid = N_PAD // LBLK
    lab_spec = pl.BlockSpec((LBLK, F), lambda l, b: (l, 0))
    lab_spec_hi = pl.BlockSpec((LBLK, F), lambda l, b: (l + nl_grid, 0))
    full2 = lambda r, c: pl.BlockSpec((r, c), lambda l, b: (0, 0))
    e_relu = pl.pallas_call(
        _att_kernel,
        grid=(nl_grid, B),
        in_specs=[pl.BlockSpec((1, S, HID), lambda l, b: (b, 0, 0)),
                  pl.BlockSpec((1, S, IN_F), lambda l, b: (b, 0, 0)),
                  lab_spec, full2(F, HID), full2(1, HID)],
        out_specs=pl.BlockSpec((1, LBLK, IN_F), lambda l, b: (b, l, 0)),
        out_shape=jax.ShapeDtypeStruct((B, N_PAD, IN_F), f32),
        scratch_shapes=[pltpu.VMEM((LBLK, HID), f32)],
    )(h, x, nodes_p, W2, b2.reshape(1, HID))

    for bg, wg in ((bg0, Wg1), (bg1, Wg2)):
        z = _sc_pass(y2, src2d, dst2d, ones_blk, zeros_blk, gather=True)
        y2 = pl.pallas_call(
            _layer_kernel,
            grid=(nb_grid,),
            in_specs=[row_spec, row_spec_hi, row_spec, row_spec,
                      mat_spec(1, F), mat_spec(F, F)],
            out_specs=row_spec,
            out_shape=jax.ShapeDtypeStruct((N_PAD, F), f32),
        )(z, z, y2, dinv, bg.reshape(1, F), wg)
    z2 = _sc_pass(y2, src2d, dst2d, ones_blk, zeros_blk, gather=True)

    out = pl.pallas_call(
        _fin_kernel,
        grid=(nl_grid, B),
        in_specs=[pl.BlockSpec((1, LBLK, IN_F), lambda l, b: (b, l, 0)),
                  lab_spec,
                  lab_spec, lab_spec_hi, lab_spec, lab_spec,
                  full2(1, F), full2(2 * F, IN_F), full2(1, IN_F)],
        out_specs=pl.BlockSpec((1, LBLK, IN_F), lambda l, b: (b, l, 0)),
        out_shape=jax.ShapeDtypeStruct((B, N_PAD, IN_F), f32),
        scratch_shapes=[pltpu.VMEM((LBLK, IN_F), f32)],
    )(e_relu, nodes_p, z2, z2, y2, dinv,
      bg2.reshape(1, F), Wp, bp.reshape(1, IN_F))
    return out[:, :N_NODES, :]
